# Initial kernel scaffold; baseline (speedup 1.0000x reference)
import jax
import jax.numpy as jnp
from jax.experimental import pallas as pl


def kernel(x, emb_table, W, b):
    raise NotImplementedError("write your pallas kernel here")


# speedup vs baseline: 4.8171x; 4.8171x over previous
import jax
import jax.numpy as jnp
from jax import lax
from jax.experimental import pallas as pl
from jax.experimental.pallas import tpu as pltpu
from jax.experimental.pallas import tpu_sc as plsc

B, L, E, O = 16384, 200, 3, 5
N = B * L
NC, NS = 2, 16
NW = NC * NS
TOK = N // NW
CHUNK = 2048
ITERS = CHUNK // 16
N_CHUNKS = TOK // CHUNK


def _build_lut(emb_v, w_v, b_v, lut_v):
    for off in (0, 16):
        p = jnp.minimum(lax.iota(jnp.int32, 16) + off, O * O - 1)
        i_v = p // O
        o_v = p % O
        acc = plsc.load_gather(b_v, [o_v])
        for d in range(E):
            dd = jnp.full((16,), d, jnp.int32)
            ev = plsc.load_gather(emb_v, [i_v, dd])
            wv = plsc.load_gather(w_v, [o_v, dd])
            acc = acc + ev * wv
        plsc.store_scatter(lut_v, [i_v, o_v], acc)


def _sc_body(emb_hbm, w_hbm, b_hbm, x_hbm, out_hbm,
             emb_v, w_v, b_v, lut_v, idx_v, out_v,
             sem_in0, sem_in1, sem_out0, sem_out1):
    c = lax.axis_index("c")
    s = lax.axis_index("s")
    wid = c * NS + s
    span0 = wid * TOK

    pltpu.sync_copy(emb_hbm, emb_v)
    pltpu.sync_copy(w_hbm, w_v)
    pltpu.sync_copy(b_hbm, b_v)
    _build_lut(emb_v, w_v, b_v, lut_v)

    sem_in = (sem_in0, sem_in1)
    sem_out = (sem_out0, sem_out1)
    pidx = lax.iota(jnp.int32, 16) * O
    o_consts = [jnp.full((16,), o, jnp.int32) for o in range(O)]

    def compute(idx_ref, out_ref):
        def it_body(it, carry):
            tb = it * 16
            xg = idx_ref[pl.ds(tb, 16)]
            base_v = pidx + tb * O
            for o in range(O):
                v = plsc.load_gather(lut_v, [xg, o_consts[o]])
                plsc.store_scatter(out_ref, [base_v + o], v)
            return carry
        lax.fori_loop(0, ITERS, it_body, 0)

    def start_in(i):
        tok0 = span0 + i * CHUNK
        return pltpu.async_copy(
            x_hbm.at[pl.ds(tok0, CHUNK)], idx_v.at[i % 2], sem_in[i % 2])

    h_in = [None, None]
    h_out = [None, None]
    h_in[0] = start_in(0)
    for i in range(N_CHUNKS):
        bf = i % 2
        if i + 1 < N_CHUNKS:
            h_in[(i + 1) % 2] = start_in(i + 1)
        h_in[bf].wait()
        if i >= 2:
            h_out[bf].wait()
        compute(idx_v.at[bf], out_v.at[bf])
        tok0 = span0 + i * CHUNK
        h_out[bf] = pltpu.async_copy(
            out_v.at[bf], out_hbm.at[pl.ds(tok0 * O, CHUNK * O)], sem_out[bf])
    h_out[(N_CHUNKS - 2) % 2].wait()
    h_out[(N_CHUNKS - 1) % 2].wait()


def _make_sc_kernel(interpret=False):
    return pl.kernel(
        _sc_body,
        out_type=jax.ShapeDtypeStruct((N * O,), jnp.float32),
        mesh=plsc.VectorSubcoreMesh(core_axis_name="c", subcore_axis_name="s",
                                    num_cores=NC, num_subcores=NS),
        compiler_params=pltpu.CompilerParams(
            use_tc_tiling_on_sc=False, needs_layout_passes=False),
        scratch_types=[
            pltpu.VMEM((O, E), jnp.float32),
            pltpu.VMEM((O, E), jnp.float32),
            pltpu.VMEM((O,), jnp.float32),
            pltpu.VMEM((O, O), jnp.float32),
            pltpu.VMEM((2, CHUNK), jnp.int32),
            pltpu.VMEM((2, CHUNK * O), jnp.float32),
            pltpu.SemaphoreType.DMA,
            pltpu.SemaphoreType.DMA,
            pltpu.SemaphoreType.DMA,
            pltpu.SemaphoreType.DMA,
        ],
        interpret=interpret,
    )


_sc_embed_mlp = _make_sc_kernel()


def kernel(x, emb_table, W, b):
    x1 = x.astype(jnp.int32).reshape(N)
    out = _sc_embed_mlp(emb_table, W, b, x1)
    return out.reshape(B, L, O)


# speedup vs baseline: 6.1570x; 1.2782x over previous
import jax
import jax.numpy as jnp
from jax import lax
from jax.experimental import pallas as pl
from jax.experimental.pallas import tpu as pltpu
from jax.experimental.pallas import tpu_sc as plsc

B, L, E, O = 16384, 200, 3, 5
N = B * L
NC, NS = 2, 16
NW = NC * NS
TOK = N // NW
CHUNK = 2048
ITERS = CHUNK // 16
N_CHUNKS = TOK // CHUNK


def _build_lut(emb_v, w_v, b_v, lut_v):
    for off in (0, 16):
        p = jnp.minimum(lax.iota(jnp.int32, 16) + off, O * O - 1)
        i_v = p // O
        o_v = p % O
        acc = plsc.load_gather(b_v, [o_v])
        for d in range(E):
            dd = jnp.full((16,), d, jnp.int32)
            ev = plsc.load_gather(emb_v, [i_v, dd])
            wv = plsc.load_gather(w_v, [o_v, dd])
            acc = acc + ev * wv
        plsc.store_scatter(lut_v, [i_v, o_v], acc)


def _sc_body(emb_hbm, w_hbm, b_hbm, x_hbm, out_hbm,
             emb_v, w_v, b_v, lut_v, idx_v, out_v,
             sem_in0, sem_in1, sem_out0, sem_out1):
    c = lax.axis_index("c")
    s = lax.axis_index("s")
    wid = c * NS + s
    span0 = wid * TOK

    pltpu.sync_copy(emb_hbm, emb_v)
    pltpu.sync_copy(w_hbm, w_v)
    pltpu.sync_copy(b_hbm, b_v)
    _build_lut(emb_v, w_v, b_v, lut_v)

    sem_in = (sem_in0, sem_in1)
    sem_out = (sem_out0, sem_out1)
    o_consts = [jnp.full((16,), o, jnp.int32) for o in range(O)]

    lane = lax.iota(jnp.int32, 16)

    def compute(idx_ref, out_ref):
        def it_body(it, carry):
            tb = it * 16
            xg = idx_ref[pl.ds(tb, 16)]
            t_v = lane + tb
            for o in range(O):
                v = plsc.load_gather(lut_v, [xg, o_consts[o]])
                plsc.store_scatter(out_ref, [t_v, o_consts[o]], v)
            return carry
        lax.fori_loop(0, ITERS, it_body, 0)

    def start_in(i):
        tok0 = span0 + i * CHUNK
        return pltpu.async_copy(
            x_hbm.at[pl.ds(tok0, CHUNK)], idx_v.at[i % 2], sem_in[i % 2])

    h_in = [None, None]
    h_out = [None, None]
    h_in[0] = start_in(0)
    for i in range(N_CHUNKS):
        bf = i % 2
        if i + 1 < N_CHUNKS:
            h_in[(i + 1) % 2] = start_in(i + 1)
        h_in[bf].wait()
        if i >= 2:
            h_out[bf].wait()
        compute(idx_v.at[bf], out_v.at[bf])
        tok0 = span0 + i * CHUNK
        h_out[bf] = pltpu.async_copy(
            out_v.at[bf], out_hbm.at[pl.ds(tok0, CHUNK), :], sem_out[bf])
    h_out[(N_CHUNKS - 2) % 2].wait()
    h_out[(N_CHUNKS - 1) % 2].wait()


def _make_sc_kernel(interpret=False):
    return pl.kernel(
        _sc_body,
        out_type=jax.ShapeDtypeStruct((N, O), jnp.float32),
        mesh=plsc.VectorSubcoreMesh(core_axis_name="c", subcore_axis_name="s",
                                    num_cores=NC, num_subcores=NS),
        compiler_params=pltpu.CompilerParams(
            use_tc_tiling_on_sc=False, needs_layout_passes=False),
        scratch_types=[
            pltpu.VMEM((O, E), jnp.float32),
            pltpu.VMEM((O, E), jnp.float32),
            pltpu.VMEM((O,), jnp.float32),
            pltpu.VMEM((O, O), jnp.float32),
            pltpu.VMEM((2, CHUNK), jnp.int32),
            pltpu.VMEM((2, CHUNK, O), jnp.float32),
            pltpu.SemaphoreType.DMA,
            pltpu.SemaphoreType.DMA,
            pltpu.SemaphoreType.DMA,
            pltpu.SemaphoreType.DMA,
        ],
        interpret=interpret,
    )


_sc_embed_mlp = _make_sc_kernel()


def kernel(x, emb_table, W, b):
    x1 = x.astype(jnp.int32).reshape(N)
    out = _sc_embed_mlp(emb_table, W, b, x1)
    return out.reshape(B, L, O)


# speedup vs baseline: 6.1802x; 1.0038x over previous
import jax
import jax.numpy as jnp
from jax import lax
from jax.experimental import pallas as pl
from jax.experimental.pallas import tpu as pltpu
from jax.experimental.pallas import tpu_sc as plsc

B, L, E, O = 16384, 200, 3, 5
N = B * L
NC, NS = 2, 16
NW = NC * NS
ROWS = B // NW
CHUNK_R = 16
CHUNK = CHUNK_R * L
ITERS = CHUNK // 16
N_CHUNKS = ROWS // CHUNK_R


def _build_lut(emb_v, w_v, b_v, lut_v):
    for off in (0, 16):
        p = jnp.minimum(lax.iota(jnp.int32, 16) + off, O * O - 1)
        i_v = p // O
        o_v = p % O
        acc = plsc.load_gather(b_v, [o_v])
        for d in range(E):
            dd = jnp.full((16,), d, jnp.int32)
            ev = plsc.load_gather(emb_v, [i_v, dd])
            wv = plsc.load_gather(w_v, [o_v, dd])
            acc = acc + ev * wv
        plsc.store_scatter(lut_v, [i_v, o_v], acc)


def _sc_body(emb_hbm, w_hbm, b_hbm, x_hbm, out_hbm,
             emb_v, w_v, b_v, lut_v, idx_v, out_v,
             sem_in0, sem_in1, sem_out0, sem_out1):
    c = lax.axis_index("c")
    s = lax.axis_index("s")
    wid = c * NS + s
    row0 = wid * ROWS

    pltpu.sync_copy(emb_hbm, emb_v)
    pltpu.sync_copy(w_hbm, w_v)
    pltpu.sync_copy(b_hbm, b_v)
    _build_lut(emb_v, w_v, b_v, lut_v)

    sem_in = (sem_in0, sem_in1)
    sem_out = (sem_out0, sem_out1)
    o_consts = [jnp.full((16,), o, jnp.int32) for o in range(O)]

    lane = lax.iota(jnp.int32, 16)
    zero_v = jnp.zeros((16,), jnp.int32)

    def compute(idx_ref, out_ref):
        def it_body(it, carry):
            tb = it * 16
            xg = idx_ref[pl.ds(tb, 16)]
            t_v = lane + tb
            for o in range(O):
                v = plsc.load_gather(lut_v, [xg, o_consts[o]])
                plsc.store_scatter(out_ref, [zero_v, t_v, o_consts[o]], v)
            return carry
        lax.fori_loop(0, ITERS, it_body, 0)

    def start_in(i):
        tok0 = (row0 + i * CHUNK_R) * L
        return pltpu.async_copy(
            x_hbm.at[pl.ds(tok0, CHUNK)], idx_v.at[i % 2], sem_in[i % 2])

    h_in = [None, None]
    h_out = [None, None]
    h_in[0] = start_in(0)
    for i in range(N_CHUNKS):
        bf = i % 2
        if i + 1 < N_CHUNKS:
            h_in[(i + 1) % 2] = start_in(i + 1)
        h_in[bf].wait()
        if i >= 2:
            h_out[bf].wait()
        compute(idx_v.at[bf], out_v.at[bf])
        r0 = row0 + i * CHUNK_R
        h_out[bf] = pltpu.async_copy(
            out_v.at[bf], out_hbm.at[pl.ds(r0, CHUNK_R), :, :], sem_out[bf])
    h_out[(N_CHUNKS - 2) % 2].wait()
    h_out[(N_CHUNKS - 1) % 2].wait()


def _make_sc_kernel(interpret=False):
    return pl.kernel(
        _sc_body,
        out_type=jax.ShapeDtypeStruct((B, L, O), jnp.float32),
        mesh=plsc.VectorSubcoreMesh(core_axis_name="c", subcore_axis_name="s",
                                    num_cores=NC, num_subcores=NS),
        compiler_params=pltpu.CompilerParams(
            use_tc_tiling_on_sc=False, needs_layout_passes=False),
        scratch_types=[
            pltpu.VMEM((O, E), jnp.float32),
            pltpu.VMEM((O, E), jnp.float32),
            pltpu.VMEM((O,), jnp.float32),
            pltpu.VMEM((O, O), jnp.float32),
            pltpu.VMEM((2, CHUNK), jnp.int32),
            pltpu.VMEM((2, CHUNK_R, L, O), jnp.float32),
            pltpu.SemaphoreType.DMA,
            pltpu.SemaphoreType.DMA,
            pltpu.SemaphoreType.DMA,
            pltpu.SemaphoreType.DMA,
        ],
        interpret=interpret,
    )


_sc_embed_mlp = _make_sc_kernel()


def kernel(x, emb_table, W, b):
    x1 = x.astype(jnp.int32).reshape(N)
    return _sc_embed_mlp(emb_table, W, b, x1)


# speedup vs baseline: 33.0678x; 5.3506x over previous
import jax
import jax.numpy as jnp
from jax import lax
from jax.experimental import pallas as pl
from jax.experimental.pallas import tpu as pltpu
from jax.experimental.pallas import tpu_sc as plsc

B, L, E, O = 16384, 200, 3, 5
NC, NS = 2, 16
NW = NC * NS
ROWS = B // NW
BB = 128
NBB = ROWS // BB
LC = 25
NLC = L // LC


def _build_lut(emb_v, w_v, b_v, lut_v):
    for off in (0, 16):
        p = jnp.minimum(lax.iota(jnp.int32, 16) + off, O * O - 1)
        i_v = p // O
        o_v = p % O
        acc = plsc.load_gather(b_v, [o_v])
        for d in range(E):
            dd = jnp.full((16,), d, jnp.int32)
            ev = plsc.load_gather(emb_v, [i_v, dd])
            wv = plsc.load_gather(w_v, [o_v, dd])
            acc = acc + ev * wv
        plsc.store_scatter(lut_v, [i_v, o_v], acc)


def _sc_body(emb_hbm, w_hbm, b_hbm, x_hbm, out_hbm,
             emb_v, w_v, b_v, lut_v, x_st, out_st,
             sem_x0, sem_x1, sem_o0, sem_o1):
    c = lax.axis_index("c")
    s = lax.axis_index("s")
    wid = c * NS + s
    b0 = wid * ROWS

    pltpu.sync_copy(emb_hbm, emb_v)
    pltpu.sync_copy(w_hbm, w_v)
    pltpu.sync_copy(b_hbm, b_v)
    _build_lut(emb_v, w_v, b_v, lut_v)

    sem_x = (sem_x0, sem_x1)
    sem_o = (sem_o0, sem_o1)
    lane = lax.iota(jnp.int32, 16)
    o_consts = [jnp.full((16,), o, jnp.int32) for o in range(O)]

    def compute(xref, oref, l0):
        def l_body(dl, carry):
            l_v = jnp.full((16,), l0, jnp.int32) + dl
            for g in range(BB // 16):
                b_v = lane + g * 16
                xg = plsc.load_gather(xref, [b_v, l_v])
                for o in range(O):
                    v = plsc.load_gather(lut_v, [xg, o_consts[o]])
                    oref[o, dl, pl.ds(g * 16, 16)] = v
            return carry
        lax.fori_loop(0, LC, l_body, 0)

    def start_x(i):
        return pltpu.async_copy(
            x_hbm.at[pl.ds(b0 + i * BB, BB), :], x_st.at[i % 2], sem_x[i % 2])

    hx = [None, None]
    ho = [None, None]
    hx[0] = start_x(0)
    k = 0
    for bb in range(NBB):
        if bb + 1 < NBB:
            hx[(bb + 1) % 2] = start_x(bb + 1)
        hx[bb % 2].wait()
        for lc in range(NLC):
            buf = k % 2
            if k >= 2:
                ho[buf].wait()
            compute(x_st.at[bb % 2], out_st.at[buf], lc * LC)
            ho[buf] = pltpu.async_copy(
                out_st.at[buf],
                out_hbm.at[:, pl.ds(lc * LC, LC), pl.ds(b0 + bb * BB, BB)],
                sem_o[buf])
            k += 1
    ho[0].wait()
    ho[1].wait()


def _make_sc_kernel(interpret=False):
    return pl.kernel(
        _sc_body,
        out_type=jax.ShapeDtypeStruct((O, L, B), jnp.float32),
        mesh=plsc.VectorSubcoreMesh(core_axis_name="c", subcore_axis_name="s",
                                    num_cores=NC, num_subcores=NS),
        compiler_params=pltpu.CompilerParams(
            use_tc_tiling_on_sc=False, needs_layout_passes=False),
        scratch_types=[
            pltpu.VMEM((O, E), jnp.float32),
            pltpu.VMEM((O, E), jnp.float32),
            pltpu.VMEM((O,), jnp.float32),
            pltpu.VMEM((O, O), jnp.float32),
            pltpu.VMEM((2, BB, L), jnp.int32),
            pltpu.VMEM((2, O, LC, BB), jnp.float32),
            pltpu.SemaphoreType.DMA,
            pltpu.SemaphoreType.DMA,
            pltpu.SemaphoreType.DMA,
            pltpu.SemaphoreType.DMA,
        ],
        interpret=interpret,
    )


_sc_embed_mlp = _make_sc_kernel()


def kernel(x, emb_table, W, b):
    out_t = _sc_embed_mlp(emb_table, W, b, x.astype(jnp.int32))
    return jnp.transpose(out_t, (2, 1, 0))


# speedup vs baseline: 33.2205x; 1.0046x over previous
import jax
import jax.numpy as jnp
from jax import lax
from jax.experimental import pallas as pl
from jax.experimental.pallas import tpu as pltpu
from jax.experimental.pallas import tpu_sc as plsc

B, L, E, O = 16384, 200, 3, 5
NC, NS = 2, 16
NW = NC * NS
ROWS = B // NW
BB = 128
NBB = ROWS // BB
LC = 50
NLC = L // LC
LU = 2


def _build_lut(emb_v, w_v, b_v, lut_v):
    for off in (0, 16):
        p = jnp.minimum(lax.iota(jnp.int32, 16) + off, O * O - 1)
        i_v = p // O
        o_v = p % O
        acc = plsc.load_gather(b_v, [o_v])
        for d in range(E):
            dd = jnp.full((16,), d, jnp.int32)
            ev = plsc.load_gather(emb_v, [i_v, dd])
            wv = plsc.load_gather(w_v, [o_v, dd])
            acc = acc + ev * wv
        plsc.store_scatter(lut_v, [i_v, o_v], acc)


def _sc_body(emb_hbm, w_hbm, b_hbm, x_hbm, out_hbm,
             emb_v, w_v, b_v, lut_v, x_st, out_st,
             sem_x0, sem_x1, sem_o0, sem_o1):
    c = lax.axis_index("c")
    s = lax.axis_index("s")
    wid = c * NS + s
    b0 = wid * ROWS

    pltpu.sync_copy(emb_hbm, emb_v)
    pltpu.sync_copy(w_hbm, w_v)
    pltpu.sync_copy(b_hbm, b_v)
    _build_lut(emb_v, w_v, b_v, lut_v)

    sem_x = (sem_x0, sem_x1)
    sem_o = (sem_o0, sem_o1)
    lane = lax.iota(jnp.int32, 16)
    o_consts = [jnp.full((16,), o, jnp.int32) for o in range(O)]

    def compute(xref, oref, l0):
        def l_body(it, carry):
            for u in range(LU):
                dl = it * LU + u
                l_v = jnp.full((16,), l0, jnp.int32) + dl
                for g in range(BB // 16):
                    b_v = lane + g * 16
                    xg = plsc.load_gather(xref, [b_v, l_v])
                    for o in range(O):
                        v = plsc.load_gather(lut_v, [xg, o_consts[o]])
                        oref[o, dl, pl.ds(g * 16, 16)] = v
            return carry
        lax.fori_loop(0, LC // LU, l_body, 0)

    def start_x(i):
        return pltpu.async_copy(
            x_hbm.at[pl.ds(b0 + i * BB, BB), :], x_st.at[i % 2], sem_x[i % 2])

    hx = [None, None]
    ho = [None, None]
    hx[0] = start_x(0)
    k = 0
    for bb in range(NBB):
        if bb + 1 < NBB:
            hx[(bb + 1) % 2] = start_x(bb + 1)
        hx[bb % 2].wait()
        for lc in range(NLC):
            buf = k % 2
            if k >= 2:
                ho[buf].wait()
            compute(x_st.at[bb % 2], out_st.at[buf], lc * LC)
            ho[buf] = pltpu.async_copy(
                out_st.at[buf],
                out_hbm.at[:, pl.ds(lc * LC, LC), pl.ds(b0 + bb * BB, BB)],
                sem_o[buf])
            k += 1
    ho[0].wait()
    ho[1].wait()


def _make_sc_kernel(interpret=False):
    return pl.kernel(
        _sc_body,
        out_type=jax.ShapeDtypeStruct((O, L, B), jnp.float32),
        mesh=plsc.VectorSubcoreMesh(core_axis_name="c", subcore_axis_name="s",
                                    num_cores=NC, num_subcores=NS),
        compiler_params=pltpu.CompilerParams(
            use_tc_tiling_on_sc=False, needs_layout_passes=False),
        scratch_types=[
            pltpu.VMEM((O, E), jnp.float32),
            pltpu.VMEM((O, E), jnp.float32),
            pltpu.VMEM((O,), jnp.float32),
            pltpu.VMEM((O, O), jnp.float32),
            pltpu.VMEM((2, BB, L), jnp.int32),
            pltpu.VMEM((2, O, LC, BB), jnp.float32),
            pltpu.SemaphoreType.DMA,
            pltpu.SemaphoreType.DMA,
            pltpu.SemaphoreType.DMA,
            pltpu.SemaphoreType.DMA,
        ],
        interpret=interpret,
    )


_sc_embed_mlp = _make_sc_kernel()


def kernel(x, emb_table, W, b):
    out_t = _sc_embed_mlp(emb_table, W, b, x.astype(jnp.int32))
    return jnp.transpose(out_t, (2, 1, 0))


# speedup vs baseline: 34.0715x; 1.0256x over previous
import jax
import jax.numpy as jnp
from jax import lax
from jax.experimental import pallas as pl
from jax.experimental.pallas import tpu as pltpu
from jax.experimental.pallas import tpu_sc as plsc

B, L, E, O = 16384, 200, 3, 5
NC, NS = 2, 16
NW = NC * NS
ROWS = B // NW
BB = 128
NBB = ROWS // BB
LC = 50
NLC = L // LC
LU = 2


def _build_lut(emb_v, w_v, b_v, lut_v):
    for off in (0, 16):
        p = jnp.minimum(lax.iota(jnp.int32, 16) + off, O * O - 1)
        i_v = p // O
        o_v = p % O
        acc = plsc.load_gather(b_v, [o_v])
        for d in range(E):
            dd = jnp.full((16,), d, jnp.int32)
            ev = plsc.load_gather(emb_v, [i_v, dd])
            wv = plsc.load_gather(w_v, [o_v, dd])
            acc = acc + ev * wv
        for r in range(16):
            rr = jnp.full((16,), r, jnp.int32)
            plsc.store_scatter(lut_v, [rr, p], acc)


def _sc_body(emb_hbm, w_hbm, b_hbm, x_hbm, out_hbm,
             emb_v, w_v, b_v, lut_v, x_st, out_st,
             sem_x0, sem_x1, sem_o0, sem_o1):
    c = lax.axis_index("c")
    s = lax.axis_index("s")
    wid = c * NS + s
    b0 = wid * ROWS

    pltpu.sync_copy(emb_hbm, emb_v)
    pltpu.sync_copy(w_hbm, w_v)
    pltpu.sync_copy(b_hbm, b_v)
    _build_lut(emb_v, w_v, b_v, lut_v)

    sem_x = (sem_x0, sem_x1)
    sem_o = (sem_o0, sem_o1)
    lane = lax.iota(jnp.int32, 16)

    def compute(xref, oref, l0):
        def l_body(it, carry):
            for u in range(LU):
                dl = it * LU + u
                l_v = jnp.full((16,), l0, jnp.int32) + dl
                for g in range(BB // 16):
                    b_v = lane + g * 16
                    xg = plsc.load_gather(xref, [b_v, l_v])
                    base = xg * O
                    for o in range(O):
                        v = plsc.load_gather(lut_v, [lane, base + o])
                        oref[o, dl, pl.ds(g * 16, 16)] = v
            return carry
        lax.fori_loop(0, LC // LU, l_body, 0)

    def start_x(i):
        return pltpu.async_copy(
            x_hbm.at[pl.ds(b0 + i * BB, BB), :], x_st.at[i % 2], sem_x[i % 2])

    hx = [None, None]
    ho = [None, None]
    hx[0] = start_x(0)
    k = 0
    for bb in range(NBB):
        if bb + 1 < NBB:
            hx[(bb + 1) % 2] = start_x(bb + 1)
        hx[bb % 2].wait()
        for lc in range(NLC):
            buf = k % 2
            if k >= 2:
                ho[buf].wait()
            compute(x_st.at[bb % 2], out_st.at[buf], lc * LC)
            ho[buf] = pltpu.async_copy(
                out_st.at[buf],
                out_hbm.at[:, pl.ds(lc * LC, LC), pl.ds(b0 + bb * BB, BB)],
                sem_o[buf])
            k += 1
    ho[0].wait()
    ho[1].wait()


def _make_sc_kernel(interpret=False):
    return pl.kernel(
        _sc_body,
        out_type=jax.ShapeDtypeStruct((O, L, B), jnp.float32),
        mesh=plsc.VectorSubcoreMesh(core_axis_name="c", subcore_axis_name="s",
                                    num_cores=NC, num_subcores=NS),
        compiler_params=pltpu.CompilerParams(
            use_tc_tiling_on_sc=False, needs_layout_passes=False),
        scratch_types=[
            pltpu.VMEM((O, E), jnp.float32),
            pltpu.VMEM((O, E), jnp.float32),
            pltpu.VMEM((O,), jnp.float32),
            pltpu.VMEM((16, O * O), jnp.float32),
            pltpu.VMEM((2, BB, L), jnp.int32),
            pltpu.VMEM((2, O, LC, BB), jnp.float32),
            pltpu.SemaphoreType.DMA,
            pltpu.SemaphoreType.DMA,
            pltpu.SemaphoreType.DMA,
            pltpu.SemaphoreType.DMA,
        ],
        interpret=interpret,
    )


_sc_embed_mlp = _make_sc_kernel()


def kernel(x, emb_table, W, b):
    out_t = _sc_embed_mlp(emb_table, W, b, x.astype(jnp.int32))
    return jnp.transpose(out_t, (2, 1, 0))
